# Initial kernel scaffold; baseline (speedup 1.0000x reference)
import jax
import jax.numpy as jnp
from jax.experimental import pallas as pl


def kernel(z, W):
    raise NotImplementedError("write your pallas kernel here")


# speedup vs baseline: 23.8105x; 23.8105x over previous
import jax
import jax.numpy as jnp
from jax.experimental import pallas as pl
from jax.experimental.pallas import tpu as pltpu

_N_E = 1024
_E_DIM = 256
_BETA = 0.25
_N_ROWS = 16384
_P = 1024
_GRID = _N_ROWS // _P


def _vq_kernel(zf_ref, w_ref, zsq_ref, esq_ref,
               enc_ref, zq_ref, idx0_ref, idx1_ref, loss_ref, perp_ref,
               counts_ref, lsum_ref):
    i = pl.program_id(0)
    zfb = zf_ref[...]
    w = w_ref[...]
    ze = jax.lax.dot_general(zfb, w, (((1,), (1,)), ((), ())))
    d = (zsq_ref[...] + esq_ref[...]) - 2.0 * ze

    iota = jax.lax.broadcasted_iota(jnp.int32, d.shape, 1)
    dmin = jnp.min(d, axis=1, keepdims=True)
    idx0 = jnp.min(jnp.where(d == dmin, iota, _N_E), axis=1, keepdims=True)
    hit0 = iota == idx0
    d2 = jnp.where(hit0, jnp.inf, d)
    dmin2 = jnp.min(d2, axis=1, keepdims=True)
    idx1 = jnp.min(jnp.where(d2 == dmin2, iota, _N_E), axis=1, keepdims=True)

    enc = hit0.astype(jnp.float32)
    enc_ref[...] = enc
    idx0_ref[...] = idx0
    idx1_ref[...] = idx1

    zq = jax.lax.dot_general(enc, w, (((1,), (0,)), ((), ())))
    zq_ref[...] = zq

    diff = zq - zfb
    part = jnp.sum(diff * diff).reshape(1, 1)
    csum = jnp.sum(enc, axis=0, keepdims=True)

    @pl.when(i == 0)
    def _init():
        lsum_ref[...] = jnp.zeros_like(lsum_ref)
        counts_ref[...] = jnp.zeros_like(counts_ref)

    lsum_ref[...] += part
    counts_ref[...] += csum

    @pl.when(i == pl.num_programs(0) - 1)
    def _fini():
        m = lsum_ref[...] / (_N_ROWS * _E_DIM)
        loss_ref[...] = m + _BETA * m
        p = counts_ref[...] * (1.0 / _N_ROWS)
        ent = jnp.sum(p * jnp.log(p + 1e-10)).reshape(1, 1)
        perp_ref[...] = jnp.exp(-ent)


def kernel(z, W):
    zp = jnp.transpose(z, (0, 2, 3, 1))
    zf = zp.reshape(-1, _E_DIM)
    z_sq = jnp.sum(zf ** 2, axis=1, keepdims=True)
    e_sq = jnp.sum(W ** 2, axis=1).reshape(1, _N_E)

    out_shapes = (
        jax.ShapeDtypeStruct((_N_ROWS, _N_E), jnp.float32),
        jax.ShapeDtypeStruct((_N_ROWS, _E_DIM), jnp.float32),
        jax.ShapeDtypeStruct((_N_ROWS, 1), jnp.int32),
        jax.ShapeDtypeStruct((_N_ROWS, 1), jnp.int32),
        jax.ShapeDtypeStruct((1, 1), jnp.float32),
        jax.ShapeDtypeStruct((1, 1), jnp.float32),
    )
    enc, zq, idx0, idx1, loss, perp = pl.pallas_call(
        _vq_kernel,
        grid=(_GRID,),
        in_specs=[
            pl.BlockSpec((_P, _E_DIM), lambda i: (i, 0)),
            pl.BlockSpec((_N_E, _E_DIM), lambda i: (0, 0)),
            pl.BlockSpec((_P, 1), lambda i: (i, 0)),
            pl.BlockSpec((1, _N_E), lambda i: (0, 0)),
        ],
        out_specs=(
            pl.BlockSpec((_P, _N_E), lambda i: (i, 0)),
            pl.BlockSpec((_P, _E_DIM), lambda i: (i, 0)),
            pl.BlockSpec((_P, 1), lambda i: (i, 0)),
            pl.BlockSpec((_P, 1), lambda i: (i, 0)),
            pl.BlockSpec((1, 1), lambda i: (0, 0)),
            pl.BlockSpec((1, 1), lambda i: (0, 0)),
        ),
        out_shape=out_shapes,
        scratch_shapes=[
            pltpu.VMEM((1, _N_E), jnp.float32),
            pltpu.VMEM((1, 1), jnp.float32),
        ],
    )(zf, W, z_sq, e_sq)

    z_q_out = jnp.transpose(zq.reshape(16, 32, 32, _E_DIM), (0, 3, 1, 2))
    return (loss.reshape(()), z_q_out, perp.reshape(()), enc, idx0, idx1)
